# baseline (speedup 1.0000x reference)
import jax
import jax.numpy as jnp
from jax import lax
from jax.experimental import pallas as pl
from jax.experimental.pallas import tpu as pltpu

B = 8192
D = 256
BM = 256
BN = 256
NRB = B // BM
NT = B // BN
SCALE = 4.539817985126859
LN2 = 0.6931471805599453


def _loss_kernel(feats_ref, comb_ref, out_ref, scf_ref):
    ft = feats_ref[...].T
    ss = jnp.sum(ft * ft, axis=0, keepdims=True)
    inv = lax.rsqrt(ss) * SCALE
    scf_ref[...] = (ft * inv).astype(jnp.bfloat16)

    rid0 = lax.broadcasted_iota(jnp.int32, (BM, 1), 0)
    cid0 = lax.broadcasted_iota(jnp.int32, (1, BN), 1)

    def fold(x):
        return x[:, 0:128] + x[:, 128:256]

    def body(rb, carry):
        m = pl.multiple_of(rb * BM, BM)
        lhs = scf_ref[:, pl.ds(m, BM)]
        rl = comb_ref[0:1, pl.ds(m, BM)].T

        acc_e = jnp.zeros((BM, 128), jnp.float32)
        acc_p = jnp.zeros((BM, 128), jnp.float32)
        acc_c = jnp.zeros((BM, 128), jnp.float32)

        for g in range(NT // 8):
            pe = pp = pc = None
            for k in range(8):
                jc = g * 8 + k
                rhs = scf_ref[:, jc * BN:(jc + 1) * BN]
                s = lax.dot_general(lhs, rhs, (((0,), (0,)), ((), ())),
                                    preferred_element_type=jnp.float32)
                ct = comb_ref[0:1, jc * BN:(jc + 1) * BN]
                eq = rl == ct
                fe = fold(jnp.exp2(s))
                fp = fold(jnp.where(eq, s, 0.0))
                fc = fold(jnp.where(eq, 1.0, 0.0))
                if k == 0:
                    pe, pp, pc = fe, fp, fc
                else:
                    pe, pp, pc = pe + fe, pp + fp, pc + fc
            acc_e = acc_e + pe
            acc_p = acc_p + pp
            acc_c = acc_c + pc

        rhs_d = scf_ref[:, pl.ds(m, BN)]
        sd = lax.dot_general(lhs, rhs_d, (((0,), (0,)), ((), ())),
                             preferred_element_type=jnp.float32)
        ldg = rid0 == cid0
        acc_e = acc_e - fold(jnp.where(ldg, jnp.exp2(sd), 0.0))
        acc_p = acc_p - fold(jnp.where(ldg, sd, 0.0))
        acc_c = acc_c - fold(jnp.where(ldg, 1.0, 0.0))

        se = jnp.sum(acc_e, axis=1, keepdims=True)
        lse = jnp.log(se)
        cnt = jnp.sum(acc_c, axis=1, keepdims=True)
        psum = jnp.sum(acc_p, axis=1, keepdims=True)
        mean = (psum * LN2 - cnt * lse) / (cnt + 1e-9)
        valid = cnt > 0
        contrib = jnp.where(valid, mean, 0.0)
        nv = jnp.where(valid, 1.0, 0.0)
        srow = jnp.sum(contrib, axis=0, keepdims=True)
        nrow = jnp.sum(nv, axis=0, keepdims=True)
        tot, nva = carry
        return (tot + srow, nva + nrow)

    zero = jnp.zeros((1, 1), jnp.float32)
    tot, nva = lax.fori_loop(0, NRB, body, (zero, zero))
    loss = -tot / jnp.maximum(nva, 1.0)
    loss = jnp.where(nva > 0, loss, 0.0)
    out_ref[...] = jnp.broadcast_to(loss, (1, 128))


def kernel(features, concept_labels, class_labels):
    comb = (concept_labels.astype(jnp.int32) * 16
            + class_labels.astype(jnp.int32))
    comb_row = comb.reshape(1, B)

    out = pl.pallas_call(
        _loss_kernel,
        grid=(1,),
        in_specs=[
            pl.BlockSpec((B, D), lambda i: (0, 0)),
            pl.BlockSpec((1, B), lambda i: (0, 0)),
        ],
        out_specs=pl.BlockSpec((1, 128), lambda i: (0, 0)),
        out_shape=jax.ShapeDtypeStruct((1, 128), jnp.float32),
        scratch_shapes=[pltpu.VMEM((D, B), jnp.bfloat16)],
        compiler_params=pltpu.CompilerParams(
            dimension_semantics=("arbitrary",),
            vmem_limit_bytes=100 * 1024 * 1024,
        ),
    )(features, comb_row)

    return out[0, 0]


# speedup vs baseline: 1.2915x; 1.2915x over previous
import jax
import jax.numpy as jnp
from jax import lax
from jax.experimental import pallas as pl
from jax.experimental.pallas import tpu as pltpu

B = 8192
D = 256
NBIN = 1024
BM = 256
BN = 256
NRB = B // BM
NT = B // BN
SCALE = 4.539817985126859
LN2 = 0.6931471805599453


def _loss_kernel(feats_ref, comb_ref, out_ref, scf_ref):
    ft = feats_ref[...].T
    ss = jnp.sum(ft * ft, axis=0, keepdims=True)
    inv = lax.rsqrt(ss) * SCALE
    scf_ref[...] = (ft * inv).astype(jnp.bfloat16)

    rid0 = lax.broadcasted_iota(jnp.int32, (BM, 1), 0)
    cid0 = lax.broadcasted_iota(jnp.int32, (1, BN), 1)
    bins = lax.broadcasted_iota(jnp.int32, (1, NBIN), 1)

    hist = jnp.zeros((1, NBIN), jnp.float32)
    for q in range(B // NBIN):
        ch = comb_ref[0:1, q * NBIN:(q + 1) * NBIN].T
        oh = jnp.where(ch == bins, 1.0, 0.0)
        hist = hist + jnp.sum(oh, axis=0, keepdims=True)

    def fold(x):
        return x[:, 0:128] + x[:, 128:256]

    def body(rb, carry):
        m = pl.multiple_of(rb * BM, BM)
        lhs = scf_ref[:, pl.ds(m, BM)]
        rl = comb_ref[0:1, pl.ds(m, BM)].T

        acc_e = jnp.zeros((BM, 128), jnp.float32)
        acc_p = jnp.zeros((BM, 128), jnp.float32)

        for g in range(NT // 4):
            pe = pp = None
            for k in range(4):
                jc = g * 4 + k
                rhs = scf_ref[:, jc * BN:(jc + 1) * BN]
                s = lax.dot_general(lhs, rhs, (((0,), (0,)), ((), ())),
                                    preferred_element_type=jnp.float32)
                ct = comb_ref[0:1, jc * BN:(jc + 1) * BN]
                eq = rl == ct
                fe = fold(jnp.exp2(s))
                fp = fold(jnp.where(eq, s, 0.0))
                if k == 0:
                    pe, pp = fe, fp
                else:
                    pe, pp = pe + fe, pp + fp
            acc_e = acc_e + pe
            acc_p = acc_p + pp

        rhs_d = scf_ref[:, pl.ds(m, BN)]
        sd = lax.dot_general(lhs, rhs_d, (((0,), (0,)), ((), ())),
                             preferred_element_type=jnp.float32)
        ldg = rid0 == cid0
        acc_e = acc_e - fold(jnp.where(ldg, jnp.exp2(sd), 0.0))
        acc_p = acc_p - fold(jnp.where(ldg, sd, 0.0))

        rowoh = jnp.where(rl == bins, 1.0, 0.0)
        cnt = lax.dot_general(rowoh, hist, (((1,), (1,)), ((), ())),
                              preferred_element_type=jnp.float32) - 1.0

        se = jnp.sum(acc_e, axis=1, keepdims=True)
        lse = jnp.log(se)
        psum = jnp.sum(acc_p, axis=1, keepdims=True)
        mean = (psum * LN2 - cnt * lse) / (cnt + 1e-9)
        valid = cnt > 0
        contrib = jnp.where(valid, mean, 0.0)
        nv = jnp.where(valid, 1.0, 0.0)
        srow = jnp.sum(contrib, axis=0, keepdims=True)
        nrow = jnp.sum(nv, axis=0, keepdims=True)
        tot, nva = carry
        return (tot + srow, nva + nrow)

    zero = jnp.zeros((1, 1), jnp.float32)
    tot, nva = lax.fori_loop(0, NRB, body, (zero, zero))
    loss = -tot / jnp.maximum(nva, 1.0)
    loss = jnp.where(nva > 0, loss, 0.0)
    out_ref[...] = jnp.broadcast_to(loss, (1, 128))


def kernel(features, concept_labels, class_labels):
    comb = (concept_labels.astype(jnp.int32) * 16
            + class_labels.astype(jnp.int32))
    comb_row = comb.reshape(1, B)

    out = pl.pallas_call(
        _loss_kernel,
        grid=(1,),
        in_specs=[
            pl.BlockSpec((B, D), lambda i: (0, 0)),
            pl.BlockSpec((1, B), lambda i: (0, 0)),
        ],
        out_specs=pl.BlockSpec((1, 128), lambda i: (0, 0)),
        out_shape=jax.ShapeDtypeStruct((1, 128), jnp.float32),
        scratch_shapes=[pltpu.VMEM((D, B), jnp.bfloat16)],
        compiler_params=pltpu.CompilerParams(
            dimension_semantics=("arbitrary",),
            vmem_limit_bytes=100 * 1024 * 1024,
        ),
    )(features, comb_row)

    return out[0, 0]


# speedup vs baseline: 1.3289x; 1.0290x over previous
import jax
import jax.numpy as jnp
from jax import lax
from jax.experimental import pallas as pl
from jax.experimental.pallas import tpu as pltpu

B = 8192
D = 256
NBIN = 1024
BM = 512
BN = 512
NRB = B // BM
NT = B // BN
SCALE = 4.539817985126859
LN2 = 0.6931471805599453


def _loss_kernel(feats_ref, comb_ref, out_ref, scf_ref):
    ft = feats_ref[...].T
    ss = jnp.sum(ft * ft, axis=0, keepdims=True)
    inv = lax.rsqrt(ss) * SCALE
    scf_ref[...] = (ft * inv).astype(jnp.bfloat16)

    rid0 = lax.broadcasted_iota(jnp.int32, (BM, 1), 0)
    cid0 = lax.broadcasted_iota(jnp.int32, (1, BN), 1)
    bins = lax.broadcasted_iota(jnp.int32, (1, NBIN), 1)

    hist = jnp.zeros((1, NBIN), jnp.float32)
    for q in range(B // NBIN):
        ch = comb_ref[0:1, q * NBIN:(q + 1) * NBIN].T
        oh = jnp.where(ch == bins, 1.0, 0.0)
        hist = hist + jnp.sum(oh, axis=0, keepdims=True)

    def fold(x):
        return (x[:, 0:128] + x[:, 128:256]) + (x[:, 256:384] + x[:, 384:512])

    def body(rb, carry):
        m = pl.multiple_of(rb * BM, BM)
        lhs = scf_ref[:, pl.ds(m, BM)]
        rl = comb_ref[0:1, pl.ds(m, BM)].T

        acc_e = jnp.zeros((BM, 128), jnp.float32)
        acc_p = jnp.zeros((BM, 128), jnp.float32)

        for g in range(NT // 4):
            pe = pp = None
            for k in range(4):
                jc = g * 4 + k
                rhs = scf_ref[:, jc * BN:(jc + 1) * BN]
                s = lax.dot_general(lhs, rhs, (((0,), (0,)), ((), ())),
                                    preferred_element_type=jnp.float32)
                ct = comb_ref[0:1, jc * BN:(jc + 1) * BN]
                eq = rl == ct
                fe = fold(jnp.exp2(s))
                fp = fold(jnp.where(eq, s, 0.0))
                if k == 0:
                    pe, pp = fe, fp
                else:
                    pe, pp = pe + fe, pp + fp
            acc_e = acc_e + pe
            acc_p = acc_p + pp

        rhs_d = scf_ref[:, pl.ds(m, BN)]
        sd = lax.dot_general(lhs, rhs_d, (((0,), (0,)), ((), ())),
                             preferred_element_type=jnp.float32)
        ldg = rid0 == cid0
        acc_e = acc_e - fold(jnp.where(ldg, jnp.exp2(sd), 0.0))
        acc_p = acc_p - fold(jnp.where(ldg, sd, 0.0))

        rowoh = jnp.where(rl == bins, 1.0, 0.0)
        cnt = lax.dot_general(rowoh, hist, (((1,), (1,)), ((), ())),
                              preferred_element_type=jnp.float32) - 1.0

        se = jnp.sum(acc_e, axis=1, keepdims=True)
        lse = jnp.log(se)
        psum = jnp.sum(acc_p, axis=1, keepdims=True)
        mean = (psum * LN2 - cnt * lse) / (cnt + 1e-9)
        valid = cnt > 0
        contrib = jnp.where(valid, mean, 0.0)
        nv = jnp.where(valid, 1.0, 0.0)
        srow = jnp.sum(contrib, axis=0, keepdims=True)
        nrow = jnp.sum(nv, axis=0, keepdims=True)
        tot, nva = carry
        return (tot + srow, nva + nrow)

    zero = jnp.zeros((1, 1), jnp.float32)
    tot, nva = lax.fori_loop(0, NRB, body, (zero, zero))
    loss = -tot / jnp.maximum(nva, 1.0)
    loss = jnp.where(nva > 0, loss, 0.0)
    out_ref[...] = jnp.broadcast_to(loss, (1, 128))


def kernel(features, concept_labels, class_labels):
    comb = (concept_labels.astype(jnp.int32) * 16
            + class_labels.astype(jnp.int32))
    comb_row = comb.reshape(1, B)

    out = pl.pallas_call(
        _loss_kernel,
        grid=(1,),
        in_specs=[
            pl.BlockSpec((B, D), lambda i: (0, 0)),
            pl.BlockSpec((1, B), lambda i: (0, 0)),
        ],
        out_specs=pl.BlockSpec((1, 128), lambda i: (0, 0)),
        out_shape=jax.ShapeDtypeStruct((1, 128), jnp.float32),
        scratch_shapes=[pltpu.VMEM((D, B), jnp.bfloat16)],
        compiler_params=pltpu.CompilerParams(
            dimension_semantics=("arbitrary",),
            vmem_limit_bytes=100 * 1024 * 1024,
        ),
    )(features, comb_row)

    return out[0, 0]


# speedup vs baseline: 1.3845x; 1.0418x over previous
import jax
import jax.numpy as jnp
from jax import lax
from jax.experimental import pallas as pl
from jax.experimental.pallas import tpu as pltpu

B = 8192
D = 256
NBIN = 1024
BM = 1024
BN = 512
NRB = B // BM
NT = B // BN
SCALE = 4.539817985126859
LN2 = 0.6931471805599453


def _loss_kernel(feats_ref, comb_ref, out_ref, scf_ref):
    ft = feats_ref[...].T
    ss = jnp.sum(ft * ft, axis=0, keepdims=True)
    inv = lax.rsqrt(ss) * SCALE
    scf_ref[...] = (ft * inv).astype(jnp.bfloat16)

    rid0 = lax.broadcasted_iota(jnp.int32, (BM, 1), 0)
    cid0 = lax.broadcasted_iota(jnp.int32, (1, BN), 1)
    bins = lax.broadcasted_iota(jnp.int32, (1, NBIN), 1)

    hist = jnp.zeros((1, NBIN), jnp.float32)
    for q in range(B // NBIN):
        ch = comb_ref[0:1, q * NBIN:(q + 1) * NBIN].T
        oh = jnp.where(ch == bins, 1.0, 0.0)
        hist = hist + jnp.sum(oh, axis=0, keepdims=True)

    def fold(x):
        return (x[:, 0:128] + x[:, 128:256]) + (x[:, 256:384] + x[:, 384:512])

    def body(rb, carry):
        m = pl.multiple_of(rb * BM, BM)
        lhs = scf_ref[:, pl.ds(m, BM)]
        rl = comb_ref[0:1, pl.ds(m, BM)].T

        acc_e = jnp.zeros((BM, 128), jnp.float32)
        acc_p = jnp.zeros((BM, 128), jnp.float32)

        for g in range(NT // 4):
            pe = pp = None
            for k in range(4):
                jc = g * 4 + k
                rhs = scf_ref[:, jc * BN:(jc + 1) * BN]
                s = lax.dot_general(lhs, rhs, (((0,), (0,)), ((), ())),
                                    preferred_element_type=jnp.float32)
                ct = comb_ref[0:1, jc * BN:(jc + 1) * BN]
                eq = rl == ct
                fe = fold(jnp.exp2(s))
                fp = fold(jnp.where(eq, s, 0.0))
                if k == 0:
                    pe, pp = fe, fp
                else:
                    pe, pp = pe + fe, pp + fp
            acc_e = acc_e + pe
            acc_p = acc_p + pp

        for h in range(BM // BN):
            rhs_d = scf_ref[:, pl.ds(m + h * BN, BN)]
            sd = lax.dot_general(lhs, rhs_d, (((0,), (0,)), ((), ())),
                                 preferred_element_type=jnp.float32)
            ldg = rid0 == (cid0 + h * BN)
            acc_e = acc_e - fold(jnp.where(ldg, jnp.exp2(sd), 0.0))
            acc_p = acc_p - fold(jnp.where(ldg, sd, 0.0))

        rowoh = jnp.where(rl == bins, 1.0, 0.0)
        cnt = lax.dot_general(rowoh, hist, (((1,), (1,)), ((), ())),
                              preferred_element_type=jnp.float32) - 1.0

        se = jnp.sum(acc_e, axis=1, keepdims=True)
        lse = jnp.log(se)
        psum = jnp.sum(acc_p, axis=1, keepdims=True)
        mean = (psum * LN2 - cnt * lse) / (cnt + 1e-9)
        valid = cnt > 0
        contrib = jnp.where(valid, mean, 0.0)
        nv = jnp.where(valid, 1.0, 0.0)
        srow = jnp.sum(contrib, axis=0, keepdims=True)
        nrow = jnp.sum(nv, axis=0, keepdims=True)
        tot, nva = carry
        return (tot + srow, nva + nrow)

    zero = jnp.zeros((1, 1), jnp.float32)
    tot, nva = lax.fori_loop(0, NRB, body, (zero, zero))
    loss = -tot / jnp.maximum(nva, 1.0)
    loss = jnp.where(nva > 0, loss, 0.0)
    out_ref[...] = jnp.broadcast_to(loss, (1, 128))


def kernel(features, concept_labels, class_labels):
    comb = (concept_labels.astype(jnp.int32) * 16
            + class_labels.astype(jnp.int32))
    comb_row = comb.reshape(1, B)

    out = pl.pallas_call(
        _loss_kernel,
        grid=(1,),
        in_specs=[
            pl.BlockSpec((B, D), lambda i: (0, 0)),
            pl.BlockSpec((1, B), lambda i: (0, 0)),
        ],
        out_specs=pl.BlockSpec((1, 128), lambda i: (0, 0)),
        out_shape=jax.ShapeDtypeStruct((1, 128), jnp.float32),
        scratch_shapes=[pltpu.VMEM((D, B), jnp.bfloat16)],
        compiler_params=pltpu.CompilerParams(
            dimension_semantics=("arbitrary",),
            vmem_limit_bytes=100 * 1024 * 1024,
        ),
    )(features, comb_row)

    return out[0, 0]
